# baseline (speedup 1.0000x reference)
import functools

import jax
import jax.numpy as jnp
from jax import lax
from jax.experimental import pallas as pl
from jax.experimental.pallas import tpu as pltpu
from jax.experimental.pallas import tpu_sc as plsc

N = 10000
E = 320000
D = 128
HD = D // 2
H = D // 2
NC = 2
NS = 16
EPW = E // NS
C = 80
NCH = EPW // C
EB = 4000
BCH = EB // C
NB = EPW // EB
NPC = N // NC
NPCP = NPC + 8
RPT = 312
ZR = 8
TAIL = NPCP - NS * RPT


def _dotb(a, b):
    return jnp.dot(a, b, preferred_element_type=jnp.float32,
                   precision=jax.lax.Precision.HIGHEST)


def _prep_body(x_ref, wn_ref, bn_ref, wm_ref, bm_ref, wep_ref, be_ref,
               nf_ref, g_ref, bp_ref):
    nf = _dotb(x_ref[...], wn_ref[...])
    nf = nf + bn_ref[...]
    nf_ref[...] = nf
    w_top = wm_ref[:D, :]
    w_bot = wm_ref[D:, :]
    c = bm_ref[...] + _dotb(be_ref[...], w_bot)
    g_ref[...] = _dotb(nf, w_top) + c
    bp_ref[...] = _dotb(wep_ref[...], w_bot)


def _prep(x, w_node, bn, wm, bm, wep, be):
    return pl.pallas_call(
        _prep_body,
        out_shape=(
            jax.ShapeDtypeStruct((N, D), jnp.float32),
            jax.ShapeDtypeStruct((N, D), jnp.float32),
            jax.ShapeDtypeStruct((8, D), jnp.float32),
        ),
    )(x, w_node, bn, wm, bm, wep, be)


def _upd_body(aggp_ref, nf_ref, wu_ref, bu_ref, wm_ref, bm_ref, wep_ref,
              be_ref, nfo_ref, g_ref, bp_ref):
    agg = jnp.concatenate([aggp_ref[0, :NPC], aggp_ref[1, :NPC]], axis=0)
    u = _dotb(agg, wu_ref[...])
    nf = jnp.maximum(u + bu_ref[...], 0.0) + nf_ref[...]
    nfo_ref[...] = nf
    w_top = wm_ref[:D, :]
    w_bot = wm_ref[D:, :]
    c = bm_ref[...] + _dotb(be_ref[...], w_bot)
    g_ref[...] = _dotb(nf, w_top) + c
    bp_ref[...] = _dotb(wep_ref[...], w_bot)


def _upd(aggp, nf, wu, bu, wm, bm, wep, be):
    return pl.pallas_call(
        _upd_body,
        out_shape=(
            jax.ShapeDtypeStruct((N, D), jnp.float32),
            jax.ShapeDtypeStruct((N, D), jnp.float32),
            jax.ShapeDtypeStruct((8, D), jnp.float32),
        ),
    )(aggp, nf, wu, bu, wm, bm, wep, be)


def _fin_body(aggp_ref, nf_ref, wu_ref, bu_ref, w1_ref, b1_ref, w2_ref,
              b2_ref, w3_ref, b3_ref, out_ref):
    agg = jnp.concatenate([aggp_ref[0, :NPC], aggp_ref[1, :NPC]], axis=0)
    u = _dotb(agg, wu_ref[...])
    nf = jnp.maximum(u + bu_ref[...], 0.0) + nf_ref[...]
    h = jnp.sum(nf, axis=0, keepdims=True) * (1.0 / N)
    h = jnp.maximum(_dotb(h, w1_ref[...]) + b1_ref[...],
                    0.0)
    h = jnp.maximum(_dotb(h, w2_ref[...]) + b2_ref[...],
                    0.0)
    out_ref[...] = _dotb(h, w3_ref[...]) + b3_ref[...]


def _fin(aggp, nf, wu, bu, w1, b1, w2, b2, w3, b3):
    return pl.pallas_call(
        _fin_body,
        out_shape=jax.ShapeDtypeStruct((1, 1), jnp.float32),
    )(aggp, nf, wu, bu, w1, b1, w2, b2, w3, b3)


def _sc_body(g_hbm, src_hbm, dst_hbm, e0_hbm, e1_hbm, b_hbm,
             out_hbm, src_v, dst_v, dsta_v, e0_v, e1_v, b_v, rows_v, m_v,
             stage_v, agg_sh, gsem, ssem):
    cid = lax.axis_index("c")
    sid = lax.axis_index("s")
    lo = cid * NPC

    pltpu.sync_copy(b_hbm, b_v)

    zero = jnp.zeros((16,), jnp.float32)

    def zrow(i, carry):
        for s in range(D // 16):
            stage_v[i, pl.ds(s * 16, 16)] = zero
        return carry

    lax.fori_loop(0, ZR, zrow, 0)
    for q in range(RPT // ZR):
        pltpu.sync_copy(stage_v, agg_sh.at[pl.ds(sid * RPT + q * ZR, ZR)])

    @pl.when(sid == 0)
    def _zero_tail():
        pltpu.sync_copy(stage_v.at[pl.ds(0, TAIL)],
                        agg_sh.at[pl.ds(NS * RPT, TAIL)])

    plsc.subcore_barrier()

    b0 = [b_v[0, pl.ds(s * 16, 16)] for s in range(D // 16)]
    b1 = [b_v[1, pl.ds(s * 16, 16)] for s in range(D // 16)]

    def compute(t, k, k2):
        def group(q, c2):
            base = q * 16
            e0g = e0_v[t, pl.ds(base, 16)]
            e1g = e1_v[t, pl.ds(base, 16)]
            for jj in range(16):
                j = base + jj
                e0 = e0g[jj]
                e1 = e1g[jj]
                for s in range(D // 16):
                    sl = pl.ds(s * 16, 16)
                    v = rows_v[k, j, sl] + e0 * b0[s] + e1 * b1[s]
                    m_v[k2, j, sl] = jnp.maximum(v, 0.0)
            return c2

        lax.fori_loop(0, C // 16, group, 0)

    def step(t):
        k = lax.rem(t, 3)
        k2 = lax.rem(t, 2)
        @pl.when(t + 2 < BCH)
        def _pref():
            kp = lax.rem(t + 2, 3)
            pltpu.async_copy(g_hbm.at[src_v.at[t + 2]], rows_v.at[kp],
                             gsem.at[kp])

        pltpu.make_async_copy(g_hbm.at[src_v.at[t]], rows_v.at[k],
                              gsem.at[k]).wait()

        @pl.when(t >= 2)
        def _drain():
            pltpu.make_async_copy(m_v.at[k2], agg_sh.at[dsta_v.at[t - 2]],
                                  ssem.at[k2]).wait()

        compute(t, k, k2)
        pltpu.async_copy(m_v.at[k2], agg_sh.at[dsta_v.at[t]], ssem.at[k2],
                         add=True)

    def block(b, carry0):
        pltpu.sync_copy(src_hbm.at[sid, b], src_v)
        pltpu.sync_copy(dst_hbm.at[sid, b], dst_v)
        pltpu.sync_copy(e0_hbm.at[sid, b], e0_v)
        pltpu.sync_copy(e1_hbm.at[sid, b], e1_v)

        def remap(r, c3):
            for q in range(C // 16):
                sl = pl.ds(q * 16, 16)
                d = dst_v[r, sl] - lo
                ok = (d >= 0) & (d < NPC)
                dsta_v[r, sl] = jnp.where(ok, d, NPC)
            return c3

        lax.fori_loop(0, BCH, remap, 0)

        for tp in range(2):
            pltpu.async_copy(g_hbm.at[src_v.at[tp]], rows_v.at[tp],
                             gsem.at[tp])

        def chunk(t, carry):
            step(t)
            return carry

        lax.fori_loop(0, BCH, chunk, 0)
        pltpu.make_async_copy(m_v.at[0], agg_sh.at[dsta_v.at[BCH - 2]],
                              ssem.at[0]).wait()
        pltpu.make_async_copy(m_v.at[1], agg_sh.at[dsta_v.at[BCH - 1]],
                              ssem.at[1]).wait()
        return carry0

    lax.fori_loop(0, NB, block, 0)
    plsc.subcore_barrier()

    for q in range(RPT // ZR):
        sl = pl.ds(sid * RPT + q * ZR, ZR)
        pltpu.sync_copy(agg_sh.at[sl], stage_v)
        pltpu.sync_copy(stage_v, out_hbm.at[cid, sl])

    @pl.when(sid == 0)
    def _flush_tail():
        sl = pl.ds(NS * RPT, TAIL)
        pltpu.sync_copy(agg_sh.at[sl], stage_v.at[pl.ds(0, TAIL)])
        pltpu.sync_copy(stage_v.at[pl.ds(0, TAIL)], out_hbm.at[cid, sl])


@functools.partial(
    pl.kernel,
    out_type=jax.ShapeDtypeStruct((NC, NPCP, D), jnp.float32),
    mesh=plsc.VectorSubcoreMesh(core_axis_name="c", subcore_axis_name="s",
                                num_cores=NC),
    scratch_types=[
        pltpu.VMEM((BCH, C), jnp.int32),
        pltpu.VMEM((BCH, C), jnp.int32),
        pltpu.VMEM((BCH, C), jnp.int32),
        pltpu.VMEM((BCH, C), jnp.float32),
        pltpu.VMEM((BCH, C), jnp.float32),
        pltpu.VMEM((8, D), jnp.float32),
        pltpu.VMEM((3, C, D), jnp.float32),
        pltpu.VMEM((2, C, D), jnp.float32),
        pltpu.VMEM((ZR, D), jnp.float32),
        pltpu.VMEM_SHARED((NPCP, D), jnp.float32),
        pltpu.SemaphoreType.DMA((3,)),
        pltpu.SemaphoreType.DMA((2,)),
    ],
)
def _sc_layer(g_hbm, src_hbm, dst_hbm, e0_hbm, e1_hbm, b_hbm,
              out_hbm, src_v, dst_v, dsta_v, e0_v, e1_v, b_v, rows_v, m_v,
              stage_v, agg_sh, gsem, ssem):
    _sc_body(g_hbm, src_hbm, dst_hbm, e0_hbm, e1_hbm, b_hbm,
             out_hbm, src_v, dst_v, dsta_v, e0_v, e1_v, b_v, rows_v, m_v,
             stage_v, agg_sh, gsem, ssem)


def kernel(x, edge_index, edge_attr, W_node, b_node, W_edge, b_edge,
           W_msg, b_msg, W_upd, b_upd, W1, b1, W2, b2, W3, b3):
    src3 = edge_index[0].reshape(NS, NB, BCH, C)
    dst3 = edge_index[1].reshape(NS, NB, BCH, C)
    e0 = edge_attr[:, 0].reshape(NS, NB, BCH, C)
    e1 = edge_attr[:, 1].reshape(NS, NB, BCH, C)
    wep = jnp.zeros((8, D), jnp.float32).at[:2, :].set(W_edge)
    bn = b_node.reshape(1, D)
    be = b_edge.reshape(1, D)

    nf, g, bp = _prep(x, W_node, bn, W_msg[0], b_msg[0].reshape(1, D),
                      wep, be)
    for i in range(W_msg.shape[0]):
        aggp = _sc_layer(g, src3, dst3, e0, e1, bp)
        if i + 1 < W_msg.shape[0]:
            nf, g, bp = _upd(aggp, nf, W_upd[i], b_upd[i].reshape(1, D),
                             W_msg[i + 1], b_msg[i + 1].reshape(1, D),
                             wep, be)
        else:
            pred = _fin(aggp, nf, W_upd[i], b_upd[i].reshape(1, D),
                        W1, b1.reshape(1, H), W2, b2.reshape(1, H),
                        W3, b3.reshape(1, 1))
    return pred


# speedup vs baseline: 3.0944x; 3.0944x over previous
import functools

import jax
import jax.numpy as jnp
from jax import lax
from jax.experimental import pallas as pl
from jax.experimental.pallas import tpu as pltpu
from jax.experimental.pallas import tpu_sc as plsc

N = 10000
E = 320000
D = 128
HD = D // 2
H = D // 2
NC = 2
NS = 16
EPW = E // NS
C = 80
NCH = EPW // C
EB = 4000
BCH = EB // C
NB = EPW // EB
NPC = N // NC
NPCP = NPC + 8
RPT = 312
ZR = 24
TAIL = NPCP - NS * RPT


def _dotb(a, b):
    return jnp.dot(a, b, preferred_element_type=jnp.float32,
                   precision=jax.lax.Precision.HIGHEST)


def _prep_body(x_ref, wn_ref, bn_ref, wm_ref, bm_ref, wep_ref, be_ref,
               nf_ref, g_ref, bp_ref):
    nf = _dotb(x_ref[...], wn_ref[...])
    nf = nf + bn_ref[...]
    nf_ref[...] = nf
    w_top = wm_ref[:D, :]
    w_bot = wm_ref[D:, :]
    c = bm_ref[...] + _dotb(be_ref[...], w_bot)
    g_ref[...] = _dotb(nf, w_top) + c
    bp_ref[...] = _dotb(wep_ref[...], w_bot)


def _prep(x, w_node, bn, wm, bm, wep, be):
    return pl.pallas_call(
        _prep_body,
        out_shape=(
            jax.ShapeDtypeStruct((N, D), jnp.float32),
            jax.ShapeDtypeStruct((N, D), jnp.float32),
            jax.ShapeDtypeStruct((8, D), jnp.float32),
        ),
    )(x, w_node, bn, wm, bm, wep, be)


def _upd_body(aggp_ref, nf_ref, wu_ref, bu_ref, wm_ref, bm_ref, wep_ref,
              be_ref, nfo_ref, g_ref, bp_ref):
    agg = jnp.concatenate([aggp_ref[0, :NPC], aggp_ref[1, :NPC]], axis=0)
    u = _dotb(agg, wu_ref[...])
    nf = jnp.maximum(u + bu_ref[...], 0.0) + nf_ref[...]
    nfo_ref[...] = nf
    w_top = wm_ref[:D, :]
    w_bot = wm_ref[D:, :]
    c = bm_ref[...] + _dotb(be_ref[...], w_bot)
    g_ref[...] = _dotb(nf, w_top) + c
    bp_ref[...] = _dotb(wep_ref[...], w_bot)


def _upd(aggp, nf, wu, bu, wm, bm, wep, be):
    return pl.pallas_call(
        _upd_body,
        out_shape=(
            jax.ShapeDtypeStruct((N, D), jnp.float32),
            jax.ShapeDtypeStruct((N, D), jnp.float32),
            jax.ShapeDtypeStruct((8, D), jnp.float32),
        ),
    )(aggp, nf, wu, bu, wm, bm, wep, be)


def _fin_body(aggp_ref, nf_ref, wu_ref, bu_ref, w1_ref, b1_ref, w2_ref,
              b2_ref, w3_ref, b3_ref, out_ref):
    agg = jnp.concatenate([aggp_ref[0, :NPC], aggp_ref[1, :NPC]], axis=0)
    u = _dotb(agg, wu_ref[...])
    nf = jnp.maximum(u + bu_ref[...], 0.0) + nf_ref[...]
    h = jnp.sum(nf, axis=0, keepdims=True) * (1.0 / N)
    h = jnp.maximum(_dotb(h, w1_ref[...]) + b1_ref[...],
                    0.0)
    h = jnp.maximum(_dotb(h, w2_ref[...]) + b2_ref[...],
                    0.0)
    out_ref[...] = _dotb(h, w3_ref[...]) + b3_ref[...]


def _fin(aggp, nf, wu, bu, w1, b1, w2, b2, w3, b3):
    return pl.pallas_call(
        _fin_body,
        out_shape=jax.ShapeDtypeStruct((1, 1), jnp.float32),
    )(aggp, nf, wu, bu, w1, b1, w2, b2, w3, b3)


def _sc_body(g_hbm, src_hbm, dst_hbm, e0_hbm, e1_hbm, b_hbm,
             out_hbm, src_v, dst_v, dsta_v, e0_v, e1_v, b_v, rows_v, m_v,
             stage_v, agg_sh, gsem0, gsem1, ssem0, ssem1):
    cid = lax.axis_index("c")
    sid = lax.axis_index("s")
    lo = cid * NPC

    pltpu.sync_copy(b_hbm, b_v)

    zero = jnp.zeros((16,), jnp.float32)

    def zrow(i, carry):
        for s in range(D // 16):
            stage_v[i, pl.ds(s * 16, 16)] = zero
        return carry

    lax.fori_loop(0, ZR, zrow, 0)
    for q in range(RPT // ZR):
        pltpu.sync_copy(stage_v, agg_sh.at[pl.ds(sid * RPT + q * ZR, ZR)])

    @pl.when(sid == 0)
    def _zero_tail():
        pltpu.sync_copy(stage_v.at[pl.ds(0, TAIL)],
                        agg_sh.at[pl.ds(NS * RPT, TAIL)])

    plsc.subcore_barrier()

    b0 = [b_v[0, pl.ds(s * 16, 16)] for s in range(D // 16)]
    b1 = [b_v[1, pl.ds(s * 16, 16)] for s in range(D // 16)]

    def compute(t, k):
        def group(q, c2):
            base = q * 16
            e0g = e0_v[t, pl.ds(base, 16)]
            e1g = e1_v[t, pl.ds(base, 16)]
            for jj in range(16):
                j = base + jj
                e0 = e0g[jj]
                e1 = e1g[jj]
                for s in range(D // 16):
                    sl = pl.ds(s * 16, 16)
                    v = rows_v[k, j, sl] + e0 * b0[s] + e1 * b1[s]
                    m_v[k, j, sl] = jnp.maximum(v, 0.0)
            return c2

        lax.fori_loop(0, C // 16, group, 0)

    gsem = (gsem0, gsem1)
    ssem = (ssem0, ssem1)

    def step(t, k):
        @pl.when(t + 1 < BCH)
        def _pref():
            pltpu.async_copy(g_hbm.at[src_v.at[t + 1]], rows_v.at[1 - k],
                             gsem[1 - k])

        pltpu.make_async_copy(g_hbm.at[src_v.at[t]], rows_v.at[k],
                              gsem[k]).wait()

        @pl.when(t >= 2)
        def _drain():
            pltpu.make_async_copy(m_v.at[k], agg_sh.at[dsta_v.at[t - 2]],
                                  ssem[k]).wait()

        compute(t, k)
        pltpu.async_copy(m_v.at[k], agg_sh.at[dsta_v.at[t]], ssem[k],
                         add=True)

    def block(b, carry0):
        pltpu.sync_copy(src_hbm.at[sid, b], src_v)
        pltpu.sync_copy(dst_hbm.at[sid, b], dst_v)
        pltpu.sync_copy(e0_hbm.at[sid, b], e0_v)
        pltpu.sync_copy(e1_hbm.at[sid, b], e1_v)

        def remap(r, c3):
            for q in range(C // 16):
                sl = pl.ds(q * 16, 16)
                d = dst_v[r, sl] - lo
                ok = (d >= 0) & (d < NPC)
                dsta_v[r, sl] = jnp.where(ok, d, NPC)
            return c3

        lax.fori_loop(0, BCH, remap, 0)

        pltpu.async_copy(g_hbm.at[src_v.at[0]], rows_v.at[0], gsem0)

        def pair(p, carry):
            step(p * 2, 0)
            step(p * 2 + 1, 1)
            return carry

        lax.fori_loop(0, BCH // 2, pair, 0)
        pltpu.make_async_copy(m_v.at[0], agg_sh.at[dsta_v.at[BCH - 2]],
                              ssem0).wait()
        pltpu.make_async_copy(m_v.at[1], agg_sh.at[dsta_v.at[BCH - 1]],
                              ssem1).wait()
        return carry0

    lax.fori_loop(0, NB, block, 0)
    plsc.subcore_barrier()

    for q in range(RPT // ZR):
        sl = pl.ds(sid * RPT + q * ZR, ZR)
        pltpu.sync_copy(agg_sh.at[sl], stage_v)
        pltpu.sync_copy(stage_v, out_hbm.at[cid, sl])

    @pl.when(sid == 0)
    def _flush_tail():
        sl = pl.ds(NS * RPT, TAIL)
        pltpu.sync_copy(agg_sh.at[sl], stage_v.at[pl.ds(0, TAIL)])
        pltpu.sync_copy(stage_v.at[pl.ds(0, TAIL)], out_hbm.at[cid, sl])


@functools.partial(
    pl.kernel,
    out_type=jax.ShapeDtypeStruct((NC, NPCP, D), jnp.float32),
    mesh=plsc.VectorSubcoreMesh(core_axis_name="c", subcore_axis_name="s",
                                num_cores=NC),
    scratch_types=[
        pltpu.VMEM((BCH, C), jnp.int32),
        pltpu.VMEM((BCH, C), jnp.int32),
        pltpu.VMEM((BCH, C), jnp.int32),
        pltpu.VMEM((BCH, C), jnp.float32),
        pltpu.VMEM((BCH, C), jnp.float32),
        pltpu.VMEM((8, D), jnp.float32),
        pltpu.VMEM((2, C, D), jnp.float32),
        pltpu.VMEM((2, C, D), jnp.float32),
        pltpu.VMEM((ZR, D), jnp.float32),
        pltpu.VMEM_SHARED((NPCP, D), jnp.float32),
        pltpu.SemaphoreType.DMA,
        pltpu.SemaphoreType.DMA,
        pltpu.SemaphoreType.DMA,
        pltpu.SemaphoreType.DMA,
    ],
)
def _sc_layer(g_hbm, src_hbm, dst_hbm, e0_hbm, e1_hbm, b_hbm,
              out_hbm, src_v, dst_v, dsta_v, e0_v, e1_v, b_v, rows_v, m_v,
              stage_v, agg_sh, gsem0, gsem1, ssem0, ssem1):
    _sc_body(g_hbm, src_hbm, dst_hbm, e0_hbm, e1_hbm, b_hbm,
             out_hbm, src_v, dst_v, dsta_v, e0_v, e1_v, b_v, rows_v, m_v,
             stage_v, agg_sh, gsem0, gsem1, ssem0, ssem1)


def kernel(x, edge_index, edge_attr, W_node, b_node, W_edge, b_edge,
           W_msg, b_msg, W_upd, b_upd, W1, b1, W2, b2, W3, b3):
    src3 = edge_index[0].reshape(NS, NB, BCH, C)
    dst3 = edge_index[1].reshape(NS, NB, BCH, C)
    e0 = edge_attr[:, 0].reshape(NS, NB, BCH, C)
    e1 = edge_attr[:, 1].reshape(NS, NB, BCH, C)
    wep = jnp.zeros((8, D), jnp.float32).at[:2, :].set(W_edge)
    bn = b_node.reshape(1, D)
    be = b_edge.reshape(1, D)

    nf, g, bp = _prep(x, W_node, bn, W_msg[0], b_msg[0].reshape(1, D),
                      wep, be)
    for i in range(W_msg.shape[0]):
        aggp = _sc_layer(g, src3, dst3, e0, e1, bp)
        if i + 1 < W_msg.shape[0]:
            nf, g, bp = _upd(aggp, nf, W_upd[i], b_upd[i].reshape(1, D),
                             W_msg[i + 1], b_msg[i + 1].reshape(1, D),
                             wep, be)
        else:
            pred = _fin(aggp, nf, W_upd[i], b_upd[i].reshape(1, D),
                        W1, b1.reshape(1, H), W2, b2.reshape(1, H),
                        W3, b3.reshape(1, 1))
    return pred
